# baseline (speedup 1.0000x reference)
import functools

import jax
import jax.numpy as jnp
from jax import lax
from jax.experimental import pallas as pl
from jax.experimental.pallas import tpu as pltpu
from jax.experimental.pallas import tpu_sc as plsc

_NUM_CORES = 2
_NUM_SUBCORES = 16
_NUM_WORKERS = _NUM_CORES * _NUM_SUBCORES

_CHUNK = 128

_UNIT = 2

_NBUF = 3
_AHEAD = 1


def _make_gather(num_rows: int, embed_dim: int, n_units: int):
  mesh = plsc.VectorSubcoreMesh(core_axis_name="c", subcore_axis_name="s")
  unit_rows = _UNIT * _CHUNK
  rows_per_worker = n_units * unit_rows

  def _unit_gather(table_hbm, idx_v, rows_v, gsem, u, slot):
    for k in range(_UNIT):
      pltpu.async_copy(
          table_hbm.at[idx_v.at[pl.ds(u * unit_rows + k * _CHUNK, _CHUNK)]],
          rows_v.at[slot, pl.ds(k * _CHUNK, _CHUNK)],
          gsem.at[slot],
      )

  @functools.partial(
      pl.kernel,
      mesh=mesh,
      out_type=jax.ShapeDtypeStruct((num_rows, embed_dim), jnp.float32),
      scratch_types=[
          pltpu.VMEM((rows_per_worker,), jnp.int32),
          pltpu.VMEM((_NBUF, unit_rows, embed_dim), jnp.float32),
          pltpu.SemaphoreType.DMA((_NBUF,)),
          pltpu.SemaphoreType.DMA((_NBUF,)),
      ],
  )
  def gather_kernel(table_hbm, idx_hbm, out_hbm, idx_v, rows_v, gsem, ssem):
    wid = lax.axis_index("s") * _NUM_CORES + lax.axis_index("c")
    base = wid * rows_per_worker
    pltpu.sync_copy(idx_hbm.at[pl.ds(base, rows_per_worker)], idx_v)

    for g in range(_AHEAD):
      _unit_gather(table_hbm, idx_v, rows_v, gsem, g, g)

    @pl.loop(0, n_units)
    def _(j):
      slot = lax.rem(j, _NBUF)
      g = j + _AHEAD

      @pl.when(g < n_units)
      def _():
        gslot = lax.rem(g, _NBUF)

        @pl.when(j >= _NBUF - _AHEAD)
        def _():
          pltpu.make_async_copy(
              rows_v.at[gslot], out_hbm.at[pl.ds(base, unit_rows)],
              ssem.at[gslot],
          ).wait()

        _unit_gather(table_hbm, idx_v, rows_v, gsem, g, gslot)

      pltpu.make_async_copy(
          out_hbm.at[pl.ds(base, unit_rows)], rows_v.at[slot], gsem.at[slot]
      ).wait()
      pltpu.async_copy(
          rows_v.at[slot],
          out_hbm.at[pl.ds(base + j * unit_rows, unit_rows)],
          ssem.at[slot],
      )

    for b in range(_NBUF):
      pltpu.make_async_copy(
          rows_v.at[b], out_hbm.at[pl.ds(base, unit_rows)], ssem.at[b]
      ).wait()

  return gather_kernel


def kernel(embeddings, token_ids):
  batch, hist = token_ids.shape
  vocab, embed_dim = embeddings.shape
  num_rows = batch * hist
  assert num_rows % (_NUM_WORKERS * _UNIT * _CHUNK) == 0
  n_units = num_rows // (_NUM_WORKERS * _UNIT * _CHUNK)

  idx = jnp.transpose(token_ids).astype(jnp.int32).reshape(num_rows)
  out = _make_gather(num_rows, embed_dim, n_units)(embeddings, idx)
  return jnp.transpose(out.reshape(hist, batch, embed_dim), (1, 0, 2))


# speedup vs baseline: 1.0132x; 1.0132x over previous
import functools

import jax
import jax.numpy as jnp
from jax import lax
from jax.experimental import pallas as pl
from jax.experimental.pallas import tpu as pltpu
from jax.experimental.pallas import tpu_sc as plsc

_NUM_CORES = 2
_NUM_SUBCORES = 16
_NUM_WORKERS = _NUM_CORES * _NUM_SUBCORES

_CHUNK = 128

_UNIT = 2

_NBUF = 3
_AHEAD = 2


def _make_gather(num_rows: int, embed_dim: int, n_units: int):
  mesh = plsc.VectorSubcoreMesh(core_axis_name="c", subcore_axis_name="s")
  unit_rows = _UNIT * _CHUNK
  rows_per_worker = n_units * unit_rows

  def _unit_gather(table_hbm, idx_v, rows_v, gsem, u, slot):
    for k in range(_UNIT):
      pltpu.async_copy(
          table_hbm.at[idx_v.at[pl.ds(u * unit_rows + k * _CHUNK, _CHUNK)]],
          rows_v.at[slot, pl.ds(k * _CHUNK, _CHUNK)],
          gsem.at[slot],
      )

  @functools.partial(
      pl.kernel,
      mesh=mesh,
      out_type=jax.ShapeDtypeStruct((num_rows, embed_dim), jnp.float32),
      scratch_types=[
          pltpu.VMEM((rows_per_worker,), jnp.int32),
          pltpu.VMEM((_NBUF, unit_rows, embed_dim), jnp.float32),
          pltpu.SemaphoreType.DMA((_NBUF,)),
          pltpu.SemaphoreType.DMA((_NBUF,)),
      ],
  )
  def gather_kernel(table_hbm, idx_hbm, out_hbm, idx_v, rows_v, gsem, ssem):
    wid = lax.axis_index("s") * _NUM_CORES + lax.axis_index("c")
    base = wid * rows_per_worker
    pltpu.sync_copy(idx_hbm.at[pl.ds(base, rows_per_worker)], idx_v)

    for g in range(_AHEAD):
      _unit_gather(table_hbm, idx_v, rows_v, gsem, g, g)

    @pl.loop(0, n_units)
    def _(j):
      slot = lax.rem(j, _NBUF)
      g = j + _AHEAD

      @pl.when(g < n_units)
      def _():
        gslot = lax.rem(g, _NBUF)

        @pl.when(j >= _NBUF - _AHEAD)
        def _():
          pltpu.make_async_copy(
              rows_v.at[gslot], out_hbm.at[pl.ds(base, unit_rows)],
              ssem.at[gslot],
          ).wait()

        _unit_gather(table_hbm, idx_v, rows_v, gsem, g, gslot)

      pltpu.make_async_copy(
          out_hbm.at[pl.ds(base, unit_rows)], rows_v.at[slot], gsem.at[slot]
      ).wait()
      pltpu.async_copy(
          rows_v.at[slot],
          out_hbm.at[pl.ds(base + j * unit_rows, unit_rows)],
          ssem.at[slot],
      )

    for b in range(_NBUF):
      pltpu.make_async_copy(
          rows_v.at[b], out_hbm.at[pl.ds(base, unit_rows)], ssem.at[b]
      ).wait()

  return gather_kernel


def kernel(embeddings, token_ids):
  batch, hist = token_ids.shape
  vocab, embed_dim = embeddings.shape
  num_rows = batch * hist
  assert num_rows % (_NUM_WORKERS * _UNIT * _CHUNK) == 0
  n_units = num_rows // (_NUM_WORKERS * _UNIT * _CHUNK)

  idx = jnp.transpose(token_ids).astype(jnp.int32).reshape(num_rows)
  out = _make_gather(num_rows, embed_dim, n_units)(embeddings, idx)
  return jnp.transpose(out.reshape(hist, batch, embed_dim), (1, 0, 2))


# speedup vs baseline: 1.0209x; 1.0076x over previous
import functools

import jax
import jax.numpy as jnp
from jax import lax
from jax.experimental import pallas as pl
from jax.experimental.pallas import tpu as pltpu
from jax.experimental.pallas import tpu_sc as plsc

_NUM_CORES = 2
_NUM_SUBCORES = 16
_NUM_WORKERS = _NUM_CORES * _NUM_SUBCORES

_CHUNK = 128

_NBUF = 7
_AHEAD = 3


def _make_gather(num_rows: int, embed_dim: int, n_chunks: int):
  mesh = plsc.VectorSubcoreMesh(core_axis_name="c", subcore_axis_name="s")
  rows_per_worker = n_chunks * _CHUNK

  @functools.partial(
      pl.kernel,
      mesh=mesh,
      out_type=jax.ShapeDtypeStruct((num_rows, embed_dim), jnp.float32),
      scratch_types=[
          pltpu.VMEM((rows_per_worker,), jnp.int32),
          pltpu.VMEM((_NBUF, _CHUNK, embed_dim), jnp.float32),
          pltpu.SemaphoreType.DMA((_NBUF,)),
          pltpu.SemaphoreType.DMA((_NBUF,)),
      ],
  )
  def gather_kernel(table_hbm, idx_hbm, out_hbm, idx_v, rows_v, gsem, ssem):
    wid = lax.axis_index("s") * _NUM_CORES + lax.axis_index("c")
    base = wid * rows_per_worker
    pltpu.sync_copy(idx_hbm.at[pl.ds(base, rows_per_worker)], idx_v)

    for g in range(_AHEAD):
      pltpu.async_copy(
          table_hbm.at[idx_v.at[pl.ds(g * _CHUNK, _CHUNK)]],
          rows_v.at[g], gsem.at[g],
      )

    @pl.loop(0, n_chunks)
    def _(j):
      slot = lax.rem(j, _NBUF)
      g = j + _AHEAD

      @pl.when(g < n_chunks)
      def _():
        gslot = lax.rem(g, _NBUF)

        @pl.when(j >= _NBUF - _AHEAD)
        def _():
          pltpu.make_async_copy(
              rows_v.at[gslot], out_hbm.at[pl.ds(base, _CHUNK)],
              ssem.at[gslot],
          ).wait()

        pltpu.async_copy(
            table_hbm.at[idx_v.at[pl.ds(g * _CHUNK, _CHUNK)]],
            rows_v.at[gslot], gsem.at[gslot],
        )

      pltpu.make_async_copy(
          table_hbm.at[idx_v.at[pl.ds(j * _CHUNK, _CHUNK)]],
          rows_v.at[slot], gsem.at[slot],
      ).wait()
      pltpu.async_copy(
          rows_v.at[slot],
          out_hbm.at[pl.ds(base + j * _CHUNK, _CHUNK)],
          ssem.at[slot],
      )

    for b in range(_NBUF):
      pltpu.make_async_copy(
          rows_v.at[b], out_hbm.at[pl.ds(base, _CHUNK)], ssem.at[b]
      ).wait()

  return gather_kernel


def kernel(embeddings, token_ids):
  batch, hist = token_ids.shape
  vocab, embed_dim = embeddings.shape
  num_rows = batch * hist
  assert num_rows % (_NUM_WORKERS * _CHUNK) == 0
  n_chunks = num_rows // (_NUM_WORKERS * _CHUNK)

  idx = jnp.transpose(token_ids).astype(jnp.int32).reshape(num_rows)
  out = _make_gather(num_rows, embed_dim, n_chunks)(embeddings, idx)
  return jnp.transpose(out.reshape(hist, batch, embed_dim), (1, 0, 2))
